# baseline (speedup 1.0000x reference)
import functools

import jax
import jax.numpy as jnp
from jax import lax
from jax.experimental import pallas as pl
from jax.experimental.pallas import tpu as pltpu
from jax.experimental.pallas import tpu_sc as plsc

N_NODES = 10000
H = 64
E = 320000
OUT_DIM = 128
IN_DIM = 128

_NC = 2
_NS = 16
_C = 128
_EROWS = 2560
_RPT = _EROWS // _NS
_NACC = 10240
_ZROWS = _NACC // _NS
_CW = 8
_NB = 8


def _mesh():
    return plsc.VectorSubcoreMesh(core_axis_name="c", subcore_axis_name="s",
                                  num_cores=_NC, num_subcores=_NS)


def _segsum_body(with_counts, ich, h, epp, edd, epd, *refs):
    if with_counts:
        (zeros64, zeros8, ones8, oA, oB, cA, cB, sidx, didx, rows, onesv,
         cntv, nbuf, acc, acc8, gsem, ssem, csem) = refs
    else:
        (cAi, cBi, zeros64, oA, oB, sidx, didx, rows,
         cntv, nbuf, acc, gsem, ssem) = refs
    nch = _RPT // ich
    g_iters = ich // _NB
    c = lax.axis_index("c")
    s = lax.axis_index("s")
    rels = (
        (0, epp, 0, 1, 0, "A", 0),
        (0, epd, 1, 0, 1, "B", 0),
        (1, edd, 0, 1, 1, "A", 1),
        (1, epd, 0, 1, 0, "B", 1),
    )
    zoff = pl.multiple_of(s * _ZROWS, 8)
    eoff = pl.multiple_of(s * _RPT, 8)

    if with_counts:
        pltpu.sync_copy(ones8, onesv)

    for rc, earr, srow, drow, tslot, outn, oslot in rels:
        out = oA if outn == "A" else oB

        @pl.when(c == rc)
        def _zero():
            pltpu.sync_copy(zeros64.at[pl.ds(zoff, _ZROWS)],
                            acc.at[pl.ds(zoff, _ZROWS)])
            if with_counts:
                pltpu.sync_copy(zeros8.at[pl.ds(zoff, _ZROWS)],
                                acc8.at[pl.ds(zoff, _ZROWS)])

        plsc.subcore_barrier()

        @pl.when(c == rc)
        def _edges(earr=earr, srow=srow, drow=drow, tslot=tslot):
            table = h.at[tslot]
            def chunk(ci, carry):
                coff = pl.multiple_of(eoff + ci * ich, 8)
                pltpu.sync_copy(earr.at[srow, pl.ds(coff, ich)], sidx)
                pltpu.sync_copy(earr.at[drow, pl.ds(coff, ich)], didx)
                for b in range(_NB):
                    pltpu.async_copy(table.at[sidx.at[b]], rows.at[b],
                                     gsem.at[b])

                def outer(g, carry2):
                    for b in range(_NB):
                        k = g * _NB + b
                        pltpu.make_async_copy(table.at[sidx.at[k]],
                                              rows.at[b], gsem.at[b]).wait()
                        pltpu.async_copy(rows.at[b], acc.at[didx.at[k]],
                                         ssem.at[b], add=True)
                        if with_counts:
                            pltpu.async_copy(onesv, acc8.at[didx.at[k]],
                                             csem.at[b], add=True)
                    for b in range(_NB):
                        k = g * _NB + b
                        pltpu.make_async_copy(rows.at[b], acc.at[didx.at[k]],
                                              ssem.at[b]).wait()
                        if with_counts:
                            pltpu.make_async_copy(
                                onesv, acc8.at[didx.at[k]],
                                csem.at[b]).wait()

                        @pl.when(g + 1 < g_iters)
                        def _next_gather(b=b, g=g):
                            kn = (g + 1) * _NB + b
                            pltpu.async_copy(table.at[sidx.at[kn]],
                                             rows.at[b], gsem.at[b])
                    return carry2

                lax.fori_loop(0, g_iters, outer, 0)
                return carry

            lax.fori_loop(0, nch, chunk, 0)

        plsc.subcore_barrier()

        @pl.when(c == rc)
        def _normalize_and_copy_out(out=out, oslot=oslot, outn=outn):
            if with_counts:
                pltpu.sync_copy(acc8.at[pl.ds(zoff, _ZROWS)], cntv)
                cout = cA if outn == "A" else cB
                pltpu.sync_copy(acc8.at[pl.ds(zoff, _ZROWS)],
                                cout.at[oslot, pl.ds(zoff, _ZROWS)])
            else:
                cin = cAi if outn == "A" else cBi
                pltpu.sync_copy(cin.at[oslot, pl.ds(zoff, _ZROWS)], cntv)

            def chunk_norm(ch, carry):
                roff = pl.multiple_of(ch * 64, 8)
                pltpu.sync_copy(acc.at[pl.ds(zoff + roff, 64)], nbuf)

                def row_norm(j, carry2):
                    cnt = plsc.load_gather(
                        cntv, [jnp.full((16,), roff + j, jnp.int32),
                               jnp.zeros((16,), jnp.int32)])
                    r = 1.0 / jnp.maximum(cnt, 1.0)
                    for c4 in range(4):
                        sl = pl.ds(c4 * 16, 16)
                        nbuf[j, sl] = nbuf[j, sl] * r
                    return carry2

                lax.fori_loop(0, 64, row_norm, 0)
                pltpu.sync_copy(nbuf,
                                out.at[oslot, pl.ds(zoff + roff, 64)])
                return carry

            lax.fori_loop(0, _ZROWS // 64, chunk_norm, 0)

        plsc.subcore_barrier()


@jax.jit
def _sc_segsum0(h, epp, edd, epd, zeros64, zeros8, ones8):
    ich = 32
    f = pl.kernel(
        functools.partial(_segsum_body, True, ich),
        out_type=[
            jax.ShapeDtypeStruct((2, _NACC, H), jnp.float32),
            jax.ShapeDtypeStruct((2, _NACC, H), jnp.float32),
            jax.ShapeDtypeStruct((2, _NACC, _CW), jnp.float32),
            jax.ShapeDtypeStruct((2, _NACC, _CW), jnp.float32),
        ],
        mesh=_mesh(),
        scratch_types=[
            pltpu.VMEM((ich, _C), jnp.int32),
            pltpu.VMEM((ich, _C), jnp.int32),
            pltpu.VMEM((_NB, _C, H), jnp.float32),
            pltpu.VMEM((_C, _CW), jnp.float32),
            pltpu.VMEM((_ZROWS, _CW), jnp.float32),
            pltpu.VMEM((64, H), jnp.float32),
            pltpu.VMEM_SHARED((_NACC, H), jnp.float32),
            pltpu.VMEM_SHARED((_NACC, _CW), jnp.float32),
            pltpu.SemaphoreType.DMA((_NB,)),
            pltpu.SemaphoreType.DMA((_NB,)),
            pltpu.SemaphoreType.DMA((_NB,)),
        ],
        compiler_params=pltpu.CompilerParams(use_tc_tiling_on_sc=False, needs_layout_passes=False),
    )
    return f(h, epp, edd, epd, zeros64, zeros8, ones8)


@jax.jit
def _sc_segsum1(h, epp, edd, epd, cA, cB, zeros64):
    ich = 40
    f = pl.kernel(
        functools.partial(_segsum_body, False, ich),
        out_type=[
            jax.ShapeDtypeStruct((2, _NACC, H), jnp.float32),
            jax.ShapeDtypeStruct((2, _NACC, H), jnp.float32),
        ],
        mesh=_mesh(),
        scratch_types=[
            pltpu.VMEM((ich, _C), jnp.int32),
            pltpu.VMEM((ich, _C), jnp.int32),
            pltpu.VMEM((_NB, _C, H), jnp.float32),
            pltpu.VMEM((_ZROWS, _CW), jnp.float32),
            pltpu.VMEM((64, H), jnp.float32),
            pltpu.VMEM_SHARED((_NACC, H), jnp.float32),
            pltpu.SemaphoreType.DMA((_NB,)),
            pltpu.SemaphoreType.DMA((_NB,)),
        ],
        compiler_params=pltpu.CompilerParams(use_tc_tiling_on_sc=False, needs_layout_passes=False),
    )
    return f(h, epp, edd, epd, cA, cB, zeros64)


_BN = 2000
_BP = _BN // 2
_NPR = _NACC // 2


def _enc(xp, W, b):
    def body(x_ref, w_ref, b_ref, o_ref):
        halves = []
        for lo in (0, IN_DIM):
            t = jnp.dot(x_ref[0][:, lo:lo + IN_DIM], w_ref[0],
                        preferred_element_type=jnp.float32) + b_ref[0]
            halves.append(jnp.maximum(t, 0.0))
        o_ref[0] = jnp.concatenate(halves, axis=1)

    return pl.pallas_call(
        body,
        grid=(2, N_NODES // _BN),
        in_specs=[
            pl.BlockSpec((1, _BP, 2 * IN_DIM), lambda t, i: (t, i, 0)),
            pl.BlockSpec((1, IN_DIM, H), lambda t, i: (t, 0, 0)),
            pl.BlockSpec((1, 1, H), lambda t, i: (t, 0, 0)),
        ],
        out_specs=pl.BlockSpec((1, _BP, 2 * H), lambda t, i: (t, i, 0)),
        out_shape=jax.ShapeDtypeStruct((2, _NPR, 2 * H), jnp.float32),
    )(xp, W, b)


def _combine(layer, sA, sB, h, Wl, bl, Wr, decW=None, decb=None):
    decode = decW is not None
    l = layer

    def body(*refs):
        if decode:
            (sa, sb, hh, wa, wb, wra, wrb, bb, dw, db, o) = refs
        else:
            (sa, sb, hh, wa, wb, wra, wrb, bb, o) = refs
        tslot = pl.program_id(0)
        a = sa[0]
        bmsg = sb[0]
        wr = wra[0, 0] + wrb[0, 0]
        bias = bb[0, tslot] + bb[0, 3 - tslot]
        halves = []
        for lo in (0, H):
            t = jnp.dot(a[:, lo:lo + H], wa[0, 0],
                        preferred_element_type=jnp.float32)
            t = t + jnp.dot(bmsg[:, lo:lo + H], wb[0, 0],
                            preferred_element_type=jnp.float32)
            t = t + jnp.dot(hh[0][:, lo:lo + H], wr,
                            preferred_element_type=jnp.float32)
            t = jnp.maximum(t + bias, 0.0)
            if decode:
                t = jnp.dot(t, dw[0], preferred_element_type=jnp.float32) \
                    + db[0]
            halves.append(t)
        o[0] = jnp.concatenate(halves, axis=1)

    in_specs = [
        pl.BlockSpec((1, _BP, 2 * H), lambda t, i: (t, i, 0)),
        pl.BlockSpec((1, _BP, 2 * H), lambda t, i: (t, i, 0)),
        pl.BlockSpec((1, _BP, 2 * H), lambda t, i: (t, i, 0)),
        pl.BlockSpec((1, 1, H, H), lambda t, i: (l, t, 0, 0)),
        pl.BlockSpec((1, 1, H, H), lambda t, i: (l, 3 - t, 0, 0)),
        pl.BlockSpec((1, 1, H, H), lambda t, i: (l, t, 0, 0)),
        pl.BlockSpec((1, 1, H, H), lambda t, i: (l, 3 - t, 0, 0)),
        pl.BlockSpec((1, 4, H), lambda t, i: (l, 0, 0)),
    ]
    args = [sA, sB, h, Wl, Wl, Wr, Wr, bl]
    if decode:
        in_specs += [
            pl.BlockSpec((1, H, OUT_DIM), lambda t, i: (t, 0, 0)),
            pl.BlockSpec((1, 1, OUT_DIM), lambda t, i: (t, 0, 0)),
        ]
        args += [decW, decb]
        out_spec = pl.BlockSpec((1, _BP, 2 * OUT_DIM), lambda t, i: (t, i, 0))
        out_shape = jax.ShapeDtypeStruct((2, N_NODES // 2, 2 * OUT_DIM),
                                         jnp.float32)
    else:
        out_spec = pl.BlockSpec((1, _BP, 2 * H), lambda t, i: (t, i, 0))
        out_shape = jax.ShapeDtypeStruct((2, _NPR, 2 * H), jnp.float32)

    return pl.pallas_call(
        body,
        grid=(2, N_NODES // _BN),
        in_specs=in_specs,
        out_specs=out_spec,
        out_shape=out_shape,
    )(*args)


def _pad_edges(ei):
    npad = _EROWS * _C - E
    tail = N_NODES + (jnp.arange(npad, dtype=jnp.int32)
                      % (_NACC - N_NODES))
    tail = jnp.broadcast_to(tail, (2, npad))
    return jnp.concatenate([ei.astype(jnp.int32), tail],
                           axis=1).reshape(2, _EROWS, _C)


def kernel(x_primal, x_dual, edge_index_pp, edge_index_dd, edge_index_pd,
           enc_p_W, enc_p_b, enc_d_W, enc_d_b, Wl, bl, Wr,
           dec_p_W, dec_p_b, dec_d_W, dec_d_b):
    epp = _pad_edges(edge_index_pp)
    edd = _pad_edges(edge_index_dd)
    epd = _pad_edges(edge_index_pd)

    zeros64 = jnp.zeros((_NACC, H), jnp.float32)
    zeros8 = jnp.zeros((_NACC, _CW), jnp.float32)
    ones8 = jnp.ones((_C, _CW), jnp.float32)

    x_st = jnp.stack([x_primal, x_dual]).reshape(2, N_NODES // 2, 2 * IN_DIM)
    encW = jnp.stack([enc_p_W, enc_d_W])
    encb = jnp.stack([enc_p_b, enc_d_b]).reshape(2, 1, H)
    h = _enc(x_st, encW, encb)

    decW = jnp.stack([dec_p_W, dec_d_W])
    decb = jnp.stack([dec_p_b, dec_d_b]).reshape(2, 1, OUT_DIM)

    def unpair(a):
        return a.reshape(2, _NACC, H)

    def pair(a):
        return a.reshape(2, _NPR, 2 * H)

    sA, sB, cA, cB = _sc_segsum0(unpair(h), epp, edd, epd,
                                 zeros64, zeros8, ones8)
    h = _combine(0, pair(sA), pair(sB), h, Wl, bl, Wr)
    sA, sB = _sc_segsum1(unpair(h), epp, edd, epd, cA, cB, zeros64)
    out = _combine(1, pair(sA), pair(sB), h, Wl, bl, Wr,
                   decW=decW, decb=decb)
    out = out.reshape(2, N_NODES, OUT_DIM)
    return (out[0], out[1])


# speedup vs baseline: 1.0280x; 1.0280x over previous
import functools

import jax
import jax.numpy as jnp
from jax import lax
from jax.experimental import pallas as pl
from jax.experimental.pallas import tpu as pltpu
from jax.experimental.pallas import tpu_sc as plsc

N_NODES = 10000
H = 64
E = 320000
OUT_DIM = 128
IN_DIM = 128

_NC = 2
_NS = 16
_C = 128
_EROWS = 2560
_RPT = _EROWS // _NS
_NACC = 10240
_ZROWS = _NACC // _NS
_CW = 8
_NB = 8


def _mesh():
    return plsc.VectorSubcoreMesh(core_axis_name="c", subcore_axis_name="s",
                                  num_cores=_NC, num_subcores=_NS)


def _segsum_body(with_counts, ich, h, epp, edd, epd, *refs):
    if with_counts:
        (zeros64, zeros8, ones8, oA, oB, cA, cB, sidx, didx, rows, onesv,
         cntv, nbuf, recips, acc, acc8, gsem, ssem, csem) = refs
    else:
        (cAi, cBi, zeros64, oA, oB, sidx, didx, rows,
         cntv, nbuf, recips, acc, gsem, ssem) = refs
    nch = _RPT // ich
    g_iters = ich // _NB
    c = lax.axis_index("c")
    s = lax.axis_index("s")
    rels = (
        (0, epp, 0, 1, 0, "A", 0),
        (0, epd, 1, 0, 1, "B", 0),
        (1, edd, 0, 1, 1, "A", 1),
        (1, epd, 0, 1, 0, "B", 1),
    )
    zoff = pl.multiple_of(s * _ZROWS, 8)
    eoff = pl.multiple_of(s * _RPT, 8)

    if with_counts:
        pltpu.sync_copy(ones8, onesv)

    for rc, earr, srow, drow, tslot, outn, oslot in rels:
        out = oA if outn == "A" else oB

        @pl.when(c == rc)
        def _zero():
            pltpu.sync_copy(zeros64.at[pl.ds(zoff, _ZROWS)],
                            acc.at[pl.ds(zoff, _ZROWS)])
            if with_counts:
                pltpu.sync_copy(zeros8.at[pl.ds(zoff, _ZROWS)],
                                acc8.at[pl.ds(zoff, _ZROWS)])

        plsc.subcore_barrier()

        @pl.when(c == rc)
        def _edges(earr=earr, srow=srow, drow=drow, tslot=tslot):
            table = h.at[tslot]
            def chunk(ci, carry):
                coff = pl.multiple_of(eoff + ci * ich, 8)
                pltpu.sync_copy(earr.at[srow, pl.ds(coff, ich)], sidx)
                pltpu.sync_copy(earr.at[drow, pl.ds(coff, ich)], didx)
                for b in range(_NB):
                    pltpu.async_copy(table.at[sidx.at[b]], rows.at[b],
                                     gsem.at[b])

                def outer(g, carry2):
                    for b in range(_NB):
                        k = g * _NB + b
                        pltpu.make_async_copy(table.at[sidx.at[k]],
                                              rows.at[b], gsem.at[b]).wait()
                        pltpu.async_copy(rows.at[b], acc.at[didx.at[k]],
                                         ssem.at[b], add=True)
                        if with_counts:
                            pltpu.async_copy(onesv, acc8.at[didx.at[k]],
                                             csem.at[b], add=True)
                    for b in range(_NB):
                        k = g * _NB + b
                        pltpu.make_async_copy(rows.at[b], acc.at[didx.at[k]],
                                              ssem.at[b]).wait()
                        if with_counts:
                            pltpu.make_async_copy(
                                onesv, acc8.at[didx.at[k]],
                                csem.at[b]).wait()

                        @pl.when(g + 1 < g_iters)
                        def _next_gather(b=b, g=g):
                            kn = (g + 1) * _NB + b
                            pltpu.async_copy(table.at[sidx.at[kn]],
                                             rows.at[b], gsem.at[b])
                    return carry2

                lax.fori_loop(0, g_iters, outer, 0)
                return carry

            lax.fori_loop(0, nch, chunk, 0)

        plsc.subcore_barrier()

        @pl.when(c == rc)
        def _normalize_and_copy_out(out=out, oslot=oslot, outn=outn):
            if with_counts:
                pltpu.sync_copy(acc8.at[pl.ds(zoff, _ZROWS)], cntv)
                cout = cA if outn == "A" else cB
                pltpu.sync_copy(acc8.at[pl.ds(zoff, _ZROWS)],
                                cout.at[oslot, pl.ds(zoff, _ZROWS)])
            else:
                cin = cAi if outn == "A" else cBi
                pltpu.sync_copy(cin.at[oslot, pl.ds(zoff, _ZROWS)], cntv)

            def recip16(g, carry):
                rowids = lax.iota(jnp.int32, 16) + g * 16
                cnt = plsc.load_gather(
                    cntv, [rowids, jnp.zeros((16,), jnp.int32)])
                recips[pl.ds(pl.multiple_of(g * 16, 8), 16)] = \
                    1.0 / jnp.maximum(cnt, 1.0)
                return carry

            lax.fori_loop(0, _ZROWS // 16, recip16, 0)

            def chunk_norm(ch, carry):
                roff = pl.multiple_of(ch * 64, 8)
                pltpu.sync_copy(acc.at[pl.ds(zoff + roff, 64)], nbuf)

                def row_norm(jj, carry2):
                    for u in range(4):
                        j = jj * 4 + u
                        r = plsc.load_gather(
                            recips, [jnp.full((16,), roff + j, jnp.int32)])
                        for c4 in range(4):
                            sl = pl.ds(c4 * 16, 16)
                            nbuf[j, sl] = nbuf[j, sl] * r
                    return carry2

                lax.fori_loop(0, 16, row_norm, 0)
                pltpu.sync_copy(nbuf,
                                out.at[oslot, pl.ds(zoff + roff, 64)])
                return carry

            lax.fori_loop(0, _ZROWS // 64, chunk_norm, 0)

        plsc.subcore_barrier()


@jax.jit
def _sc_segsum0(h, epp, edd, epd, zeros64, zeros8, ones8):
    ich = 32
    f = pl.kernel(
        functools.partial(_segsum_body, True, ich),
        out_type=[
            jax.ShapeDtypeStruct((2, _NACC, H), jnp.float32),
            jax.ShapeDtypeStruct((2, _NACC, H), jnp.float32),
            jax.ShapeDtypeStruct((2, _NACC, _CW), jnp.float32),
            jax.ShapeDtypeStruct((2, _NACC, _CW), jnp.float32),
        ],
        mesh=_mesh(),
        scratch_types=[
            pltpu.VMEM((ich, _C), jnp.int32),
            pltpu.VMEM((ich, _C), jnp.int32),
            pltpu.VMEM((_NB, _C, H), jnp.float32),
            pltpu.VMEM((_C, _CW), jnp.float32),
            pltpu.VMEM((_ZROWS, _CW), jnp.float32),
            pltpu.VMEM((64, H), jnp.float32),
            pltpu.VMEM((_ZROWS,), jnp.float32),
            pltpu.VMEM_SHARED((_NACC, H), jnp.float32),
            pltpu.VMEM_SHARED((_NACC, _CW), jnp.float32),
            pltpu.SemaphoreType.DMA((_NB,)),
            pltpu.SemaphoreType.DMA((_NB,)),
            pltpu.SemaphoreType.DMA((_NB,)),
        ],
        compiler_params=pltpu.CompilerParams(use_tc_tiling_on_sc=False, needs_layout_passes=False),
    )
    return f(h, epp, edd, epd, zeros64, zeros8, ones8)


@jax.jit
def _sc_segsum1(h, epp, edd, epd, cA, cB, zeros64):
    ich = 40
    f = pl.kernel(
        functools.partial(_segsum_body, False, ich),
        out_type=[
            jax.ShapeDtypeStruct((2, _NACC, H), jnp.float32),
            jax.ShapeDtypeStruct((2, _NACC, H), jnp.float32),
        ],
        mesh=_mesh(),
        scratch_types=[
            pltpu.VMEM((ich, _C), jnp.int32),
            pltpu.VMEM((ich, _C), jnp.int32),
            pltpu.VMEM((_NB, _C, H), jnp.float32),
            pltpu.VMEM((_ZROWS, _CW), jnp.float32),
            pltpu.VMEM((64, H), jnp.float32),
            pltpu.VMEM((_ZROWS,), jnp.float32),
            pltpu.VMEM_SHARED((_NACC, H), jnp.float32),
            pltpu.SemaphoreType.DMA((_NB,)),
            pltpu.SemaphoreType.DMA((_NB,)),
        ],
        compiler_params=pltpu.CompilerParams(use_tc_tiling_on_sc=False, needs_layout_passes=False),
    )
    return f(h, epp, edd, epd, cA, cB, zeros64)


_BN = 2000
_BP = _BN // 2
_NPR = _NACC // 2


def _enc(xp, W, b):
    def body(x_ref, w_ref, b_ref, o_ref):
        halves = []
        w16 = w_ref[0].astype(jnp.bfloat16)
        for lo in (0, IN_DIM):
            t = jnp.dot(x_ref[0][:, lo:lo + IN_DIM].astype(jnp.bfloat16),
                        w16, preferred_element_type=jnp.float32) + b_ref[0]
            halves.append(jnp.maximum(t, 0.0))
        o_ref[0] = jnp.concatenate(halves, axis=1)

    return pl.pallas_call(
        body,
        grid=(2, N_NODES // _BN),
        in_specs=[
            pl.BlockSpec((1, _BP, 2 * IN_DIM), lambda t, i: (t, i, 0)),
            pl.BlockSpec((1, IN_DIM, H), lambda t, i: (t, 0, 0)),
            pl.BlockSpec((1, 1, H), lambda t, i: (t, 0, 0)),
        ],
        out_specs=pl.BlockSpec((1, _BP, 2 * H), lambda t, i: (t, i, 0)),
        out_shape=jax.ShapeDtypeStruct((2, _NPR, 2 * H), jnp.float32),
    )(xp, W, b)


def _combine(layer, sA, sB, h, Wl, bl, Wr, decW=None, decb=None):
    decode = decW is not None
    l = layer

    def body(*refs):
        if decode:
            (sa, sb, hh, wa, wb, wra, wrb, bb, dw, db, o) = refs
        else:
            (sa, sb, hh, wa, wb, wra, wrb, bb, o) = refs
        tslot = pl.program_id(0)
        a = sa[0]
        bmsg = sb[0]
        wr = wra[0, 0] + wrb[0, 0]
        bias = bb[0, tslot] + bb[0, 3 - tslot]
        halves = []
        wa16 = wa[0, 0].astype(jnp.bfloat16)
        wb16 = wb[0, 0].astype(jnp.bfloat16)
        wr16 = wr.astype(jnp.bfloat16)
        dw16 = dw[0].astype(jnp.bfloat16) if decode else None
        for lo in (0, H):
            t = jnp.dot(a[:, lo:lo + H].astype(jnp.bfloat16), wa16,
                        preferred_element_type=jnp.float32)
            t = t + jnp.dot(bmsg[:, lo:lo + H].astype(jnp.bfloat16), wb16,
                            preferred_element_type=jnp.float32)
            t = t + jnp.dot(hh[0][:, lo:lo + H].astype(jnp.bfloat16), wr16,
                            preferred_element_type=jnp.float32)
            t = jnp.maximum(t + bias, 0.0)
            if decode:
                t = jnp.dot(t.astype(jnp.bfloat16), dw16,
                            preferred_element_type=jnp.float32) + db[0]
            halves.append(t)
        o[0] = jnp.concatenate(halves, axis=1)

    in_specs = [
        pl.BlockSpec((1, _BP, 2 * H), lambda t, i: (t, i, 0)),
        pl.BlockSpec((1, _BP, 2 * H), lambda t, i: (t, i, 0)),
        pl.BlockSpec((1, _BP, 2 * H), lambda t, i: (t, i, 0)),
        pl.BlockSpec((1, 1, H, H), lambda t, i: (l, t, 0, 0)),
        pl.BlockSpec((1, 1, H, H), lambda t, i: (l, 3 - t, 0, 0)),
        pl.BlockSpec((1, 1, H, H), lambda t, i: (l, t, 0, 0)),
        pl.BlockSpec((1, 1, H, H), lambda t, i: (l, 3 - t, 0, 0)),
        pl.BlockSpec((1, 4, H), lambda t, i: (l, 0, 0)),
    ]
    args = [sA, sB, h, Wl, Wl, Wr, Wr, bl]
    if decode:
        in_specs += [
            pl.BlockSpec((1, H, OUT_DIM), lambda t, i: (t, 0, 0)),
            pl.BlockSpec((1, 1, OUT_DIM), lambda t, i: (t, 0, 0)),
        ]
        args += [decW, decb]
        out_spec = pl.BlockSpec((1, _BP, 2 * OUT_DIM), lambda t, i: (t, i, 0))
        out_shape = jax.ShapeDtypeStruct((2, N_NODES // 2, 2 * OUT_DIM),
                                         jnp.float32)
    else:
        out_spec = pl.BlockSpec((1, _BP, 2 * H), lambda t, i: (t, i, 0))
        out_shape = jax.ShapeDtypeStruct((2, _NPR, 2 * H), jnp.float32)

    return pl.pallas_call(
        body,
        grid=(2, N_NODES // _BN),
        in_specs=in_specs,
        out_specs=out_spec,
        out_shape=out_shape,
    )(*args)


def _pad_edges(ei):
    npad = _EROWS * _C - E
    tail = N_NODES + (jnp.arange(npad, dtype=jnp.int32)
                      % (_NACC - N_NODES))
    tail = jnp.broadcast_to(tail, (2, npad))
    return jnp.concatenate([ei.astype(jnp.int32), tail],
                           axis=1).reshape(2, _EROWS, _C)


def kernel(x_primal, x_dual, edge_index_pp, edge_index_dd, edge_index_pd,
           enc_p_W, enc_p_b, enc_d_W, enc_d_b, Wl, bl, Wr,
           dec_p_W, dec_p_b, dec_d_W, dec_d_b):
    epp = _pad_edges(edge_index_pp)
    edd = _pad_edges(edge_index_dd)
    epd = _pad_edges(edge_index_pd)

    zeros64 = jnp.zeros((_NACC, H), jnp.float32)
    zeros8 = jnp.zeros((_NACC, _CW), jnp.float32)
    ones8 = jnp.ones((_C, _CW), jnp.float32)

    x_st = jnp.stack([x_primal, x_dual]).reshape(2, N_NODES // 2, 2 * IN_DIM)
    encW = jnp.stack([enc_p_W, enc_d_W])
    encb = jnp.stack([enc_p_b, enc_d_b]).reshape(2, 1, H)
    h = _enc(x_st, encW, encb)

    decW = jnp.stack([dec_p_W, dec_d_W])
    decb = jnp.stack([dec_p_b, dec_d_b]).reshape(2, 1, OUT_DIM)

    def unpair(a):
        return a.reshape(2, _NACC, H)

    def pair(a):
        return a.reshape(2, _NPR, 2 * H)

    sA, sB, cA, cB = _sc_segsum0(unpair(h), epp, edd, epd,
                                 zeros64, zeros8, ones8)
    h = _combine(0, pair(sA), pair(sB), h, Wl, bl, Wr)
    sA, sB = _sc_segsum1(unpair(h), epp, edd, epd, cA, cB, zeros64)
    out = _combine(1, pair(sA), pair(sB), h, Wl, bl, Wr,
                   decW=decW, decb=decb)
    out = out.reshape(2, N_NODES, OUT_DIM)
    return (out[0], out[1])


# speedup vs baseline: 1.0964x; 1.0665x over previous
import functools

import jax
import jax.numpy as jnp
from jax import lax
from jax.experimental import pallas as pl
from jax.experimental.pallas import tpu as pltpu
from jax.experimental.pallas import tpu_sc as plsc

N_NODES = 10000
H = 64
E = 320000
OUT_DIM = 128
IN_DIM = 128

_NC = 2
_NS = 16
_C = 128
_EROWS = 2560
_RPT = _EROWS // _NS
_NACC = 10240
_ZROWS = _NACC // _NS
_CW = 8
_NB = 8


def _mesh():
    return plsc.VectorSubcoreMesh(core_axis_name="c", subcore_axis_name="s",
                                  num_cores=_NC, num_subcores=_NS)


def _segsum_body(with_counts, ich, h, epp, edd, epd, *refs):
    if with_counts:
        (zeros64, zeros8, ones8, oA, oB, cA, cB, sidx, didx, rows, onesv,
         cntv, nbuf, recips, acc, acc8, gsem, ssem, csem, isem) = refs
    else:
        (cAi, cBi, zeros64, oA, oB, sidx, didx, rows,
         cntv, nbuf, recips, acc, gsem, ssem, isem) = refs
    nch = _RPT // ich
    g_iters = ich // _NB
    c = lax.axis_index("c")
    s = lax.axis_index("s")
    rels = (
        (0, epp, 0, 1, 0, "A", 0),
        (0, epd, 1, 0, 1, "B", 0),
        (1, edd, 0, 1, 1, "A", 1),
        (1, epd, 0, 1, 0, "B", 1),
    )
    zoff = pl.multiple_of(s * _ZROWS, 8)
    eoff = pl.multiple_of(s * _RPT, 8)

    if with_counts:
        pltpu.sync_copy(ones8, onesv)

    for rc, earr, srow, drow, tslot, outn, oslot in rels:
        out = oA if outn == "A" else oB

        @pl.when(c == rc)
        def _zero():
            pltpu.sync_copy(zeros64.at[pl.ds(zoff, _ZROWS)],
                            acc.at[pl.ds(zoff, _ZROWS)])
            if with_counts:
                pltpu.sync_copy(zeros8.at[pl.ds(zoff, _ZROWS)],
                                acc8.at[pl.ds(zoff, _ZROWS)])

        plsc.subcore_barrier()

        @pl.when(c == rc)
        def _edges(earr=earr, srow=srow, drow=drow, tslot=tslot):
            table = h.at[tslot]

            def idx_load(ci, par, sem=None):
                coff = pl.multiple_of(eoff + ci * ich, 8)
                if sem is None:
                    pltpu.sync_copy(earr.at[srow, pl.ds(coff, ich)],
                                    sidx.at[par])
                    pltpu.sync_copy(earr.at[drow, pl.ds(coff, ich)],
                                    didx.at[par])
                else:
                    pltpu.async_copy(earr.at[srow, pl.ds(coff, ich)],
                                     sidx.at[par], sem)
                    pltpu.async_copy(earr.at[drow, pl.ds(coff, ich)],
                                     didx.at[par], sem)

            def idx_drain(ci, par):
                coff = pl.multiple_of(eoff + ci * ich, 8)
                pltpu.make_async_copy(earr.at[srow, pl.ds(coff, ich)],
                                      sidx.at[par], isem).wait()
                pltpu.make_async_copy(earr.at[drow, pl.ds(coff, ich)],
                                      didx.at[par], isem).wait()

            idx_load(0, 0)
            for b in range(_NB):
                pltpu.async_copy(table.at[sidx.at[0, b]], rows.at[b],
                                 gsem.at[b])
            idx_load(1, 1, isem)

            def do_chunk(ci, cur, nxt):
                has_next = ci + 1 < nch
                for g in range(g_iters):
                    for b in range(_NB):
                        k = g * _NB + b
                        pltpu.make_async_copy(table.at[sidx.at[cur, k]],
                                              rows.at[b], gsem.at[b]).wait()
                        pltpu.async_copy(rows.at[b], acc.at[didx.at[cur, k]],
                                         ssem.at[b], add=True)
                        if with_counts:
                            pltpu.async_copy(onesv, acc8.at[didx.at[cur, k]],
                                             csem.at[b], add=True)
                    if g == g_iters - 1:
                        @pl.when(has_next)
                        def _wait_idx():
                            idx_drain(ci + 1, nxt)
                    for b in range(_NB):
                        k = g * _NB + b
                        pltpu.make_async_copy(rows.at[b],
                                              acc.at[didx.at[cur, k]],
                                              ssem.at[b]).wait()
                        if with_counts:
                            pltpu.make_async_copy(
                                onesv, acc8.at[didx.at[cur, k]],
                                csem.at[b]).wait()
                        if g + 1 < g_iters:
                            pltpu.async_copy(
                                table.at[sidx.at[cur, (g + 1) * _NB + b]],
                                rows.at[b], gsem.at[b])
                        else:
                            @pl.when(has_next)
                            def _ring_next(b=b):
                                pltpu.async_copy(table.at[sidx.at[nxt, b]],
                                                 rows.at[b], gsem.at[b])

            def chunk_pair(cp, carry):
                for par in (0, 1):
                    ci = cp * 2 + par
                    do_chunk(ci, par, 1 - par)

                    @pl.when(ci + 2 < nch)
                    def _prefetch(ci=ci, par=par):
                        idx_load(ci + 2, par, isem)
                return carry

            lax.fori_loop(0, nch // 2, chunk_pair, 0)

        plsc.subcore_barrier()

        @pl.when(c == rc)
        def _normalize_and_copy_out(out=out, oslot=oslot, outn=outn):
            if with_counts:
                pltpu.sync_copy(acc8.at[pl.ds(zoff, _ZROWS)], cntv)
                cout = cA if outn == "A" else cB
                pltpu.sync_copy(acc8.at[pl.ds(zoff, _ZROWS)],
                                cout.at[oslot, pl.ds(zoff, _ZROWS)])
            else:
                cin = cAi if outn == "A" else cBi
                pltpu.sync_copy(cin.at[oslot, pl.ds(zoff, _ZROWS)], cntv)

            def recip16(g, carry):
                rowids = lax.iota(jnp.int32, 16) + g * 16
                cnt = plsc.load_gather(
                    cntv, [rowids, jnp.zeros((16,), jnp.int32)])
                recips[pl.ds(pl.multiple_of(g * 16, 8), 16)] = \
                    1.0 / jnp.maximum(cnt, 1.0)
                return carry

            lax.fori_loop(0, _ZROWS // 16, recip16, 0)

            def chunk_norm(ch, carry):
                roff = pl.multiple_of(ch * 64, 8)
                pltpu.sync_copy(acc.at[pl.ds(zoff + roff, 64)], nbuf)

                def row_norm(jj, carry2):
                    for u in range(4):
                        j = jj * 4 + u
                        r = plsc.load_gather(
                            recips, [jnp.full((16,), roff + j, jnp.int32)])
                        for c4 in range(4):
                            sl = pl.ds(c4 * 16, 16)
                            nbuf[j, sl] = nbuf[j, sl] * r
                    return carry2

                lax.fori_loop(0, 16, row_norm, 0)
                pltpu.sync_copy(nbuf,
                                out.at[oslot, pl.ds(zoff + roff, 64)])
                return carry

            lax.fori_loop(0, _ZROWS // 64, chunk_norm, 0)

        plsc.subcore_barrier()


@jax.jit
def _sc_segsum0(h, epp, edd, epd, zeros64, zeros8, ones8):
    ich = 16
    f = pl.kernel(
        functools.partial(_segsum_body, True, ich),
        out_type=[
            jax.ShapeDtypeStruct((2, _NACC, H), jnp.float32),
            jax.ShapeDtypeStruct((2, _NACC, H), jnp.float32),
            jax.ShapeDtypeStruct((2, _NACC, _CW), jnp.float32),
            jax.ShapeDtypeStruct((2, _NACC, _CW), jnp.float32),
        ],
        mesh=_mesh(),
        scratch_types=[
            pltpu.VMEM((2, ich, _C), jnp.int32),
            pltpu.VMEM((2, ich, _C), jnp.int32),
            pltpu.VMEM((_NB, _C, H), jnp.float32),
            pltpu.VMEM((_C, _CW), jnp.float32),
            pltpu.VMEM((_ZROWS, _CW), jnp.float32),
            pltpu.VMEM((64, H), jnp.float32),
            pltpu.VMEM((_ZROWS,), jnp.float32),
            pltpu.VMEM_SHARED((_NACC, H), jnp.float32),
            pltpu.VMEM_SHARED((_NACC, _CW), jnp.float32),
            pltpu.SemaphoreType.DMA((_NB,)),
            pltpu.SemaphoreType.DMA((_NB,)),
            pltpu.SemaphoreType.DMA((_NB,)),
            pltpu.SemaphoreType.DMA,
        ],
        compiler_params=pltpu.CompilerParams(use_tc_tiling_on_sc=False, needs_layout_passes=False),
    )
    return f(h, epp, edd, epd, zeros64, zeros8, ones8)


@jax.jit
def _sc_segsum1(h, epp, edd, epd, cA, cB, zeros64):
    ich = 16
    f = pl.kernel(
        functools.partial(_segsum_body, False, ich),
        out_type=[
            jax.ShapeDtypeStruct((2, _NACC, H), jnp.float32),
            jax.ShapeDtypeStruct((2, _NACC, H), jnp.float32),
        ],
        mesh=_mesh(),
        scratch_types=[
            pltpu.VMEM((2, ich, _C), jnp.int32),
            pltpu.VMEM((2, ich, _C), jnp.int32),
            pltpu.VMEM((_NB, _C, H), jnp.float32),
            pltpu.VMEM((_ZROWS, _CW), jnp.float32),
            pltpu.VMEM((64, H), jnp.float32),
            pltpu.VMEM((_ZROWS,), jnp.float32),
            pltpu.VMEM_SHARED((_NACC, H), jnp.float32),
            pltpu.SemaphoreType.DMA((_NB,)),
            pltpu.SemaphoreType.DMA((_NB,)),
            pltpu.SemaphoreType.DMA,
        ],
        compiler_params=pltpu.CompilerParams(use_tc_tiling_on_sc=False, needs_layout_passes=False),
    )
    return f(h, epp, edd, epd, cA, cB, zeros64)


_BN = 2000
_BP = _BN // 2
_NPR = _NACC // 2


def _enc(xp, W, b):
    def body(x_ref, w_ref, b_ref, o_ref):
        halves = []
        w16 = w_ref[0].astype(jnp.bfloat16)
        for lo in (0, IN_DIM):
            t = jnp.dot(x_ref[0][:, lo:lo + IN_DIM].astype(jnp.bfloat16),
                        w16, preferred_element_type=jnp.float32) + b_ref[0]
            halves.append(jnp.maximum(t, 0.0))
        o_ref[0] = jnp.concatenate(halves, axis=1)

    return pl.pallas_call(
        body,
        grid=(2, N_NODES // _BN),
        in_specs=[
            pl.BlockSpec((1, _BP, 2 * IN_DIM), lambda t, i: (t, i, 0)),
            pl.BlockSpec((1, IN_DIM, H), lambda t, i: (t, 0, 0)),
            pl.BlockSpec((1, 1, H), lambda t, i: (t, 0, 0)),
        ],
        out_specs=pl.BlockSpec((1, _BP, 2 * H), lambda t, i: (t, i, 0)),
        out_shape=jax.ShapeDtypeStruct((2, _NPR, 2 * H), jnp.float32),
    )(xp, W, b)


def _combine(layer, sA, sB, h, Wl, bl, Wr, decW=None, decb=None):
    decode = decW is not None
    l = layer

    def body(*refs):
        if decode:
            (sa, sb, hh, wa, wb, wra, wrb, bb, dw, db, o) = refs
        else:
            (sa, sb, hh, wa, wb, wra, wrb, bb, o) = refs
        tslot = pl.program_id(0)
        a = sa[0]
        bmsg = sb[0]
        wr = wra[0, 0] + wrb[0, 0]
        bias = bb[0, tslot] + bb[0, 3 - tslot]
        halves = []
        wa16 = wa[0, 0].astype(jnp.bfloat16)
        wb16 = wb[0, 0].astype(jnp.bfloat16)
        wr16 = wr.astype(jnp.bfloat16)
        dw16 = dw[0].astype(jnp.bfloat16) if decode else None
        for lo in (0, H):
            t = jnp.dot(a[:, lo:lo + H].astype(jnp.bfloat16), wa16,
                        preferred_element_type=jnp.float32)
            t = t + jnp.dot(bmsg[:, lo:lo + H].astype(jnp.bfloat16), wb16,
                            preferred_element_type=jnp.float32)
            t = t + jnp.dot(hh[0][:, lo:lo + H].astype(jnp.bfloat16), wr16,
                            preferred_element_type=jnp.float32)
            t = jnp.maximum(t + bias, 0.0)
            if decode:
                t = jnp.dot(t.astype(jnp.bfloat16), dw16,
                            preferred_element_type=jnp.float32) + db[0]
            halves.append(t)
        o[0] = jnp.concatenate(halves, axis=1)

    in_specs = [
        pl.BlockSpec((1, _BP, 2 * H), lambda t, i: (t, i, 0)),
        pl.BlockSpec((1, _BP, 2 * H), lambda t, i: (t, i, 0)),
        pl.BlockSpec((1, _BP, 2 * H), lambda t, i: (t, i, 0)),
        pl.BlockSpec((1, 1, H, H), lambda t, i: (l, t, 0, 0)),
        pl.BlockSpec((1, 1, H, H), lambda t, i: (l, 3 - t, 0, 0)),
        pl.BlockSpec((1, 1, H, H), lambda t, i: (l, t, 0, 0)),
        pl.BlockSpec((1, 1, H, H), lambda t, i: (l, 3 - t, 0, 0)),
        pl.BlockSpec((1, 4, H), lambda t, i: (l, 0, 0)),
    ]
    args = [sA, sB, h, Wl, Wl, Wr, Wr, bl]
    if decode:
        in_specs += [
            pl.BlockSpec((1, H, OUT_DIM), lambda t, i: (t, 0, 0)),
            pl.BlockSpec((1, 1, OUT_DIM), lambda t, i: (t, 0, 0)),
        ]
        args += [decW, decb]
        out_spec = pl.BlockSpec((1, _BP, 2 * OUT_DIM), lambda t, i: (t, i, 0))
        out_shape = jax.ShapeDtypeStruct((2, N_NODES // 2, 2 * OUT_DIM),
                                         jnp.float32)
    else:
        out_spec = pl.BlockSpec((1, _BP, 2 * H), lambda t, i: (t, i, 0))
        out_shape = jax.ShapeDtypeStruct((2, _NPR, 2 * H), jnp.float32)

    return pl.pallas_call(
        body,
        grid=(2, N_NODES // _BN),
        in_specs=in_specs,
        out_specs=out_spec,
        out_shape=out_shape,
    )(*args)


def _pad_edges(ei):
    npad = _EROWS * _C - E
    tail = N_NODES + (jnp.arange(npad, dtype=jnp.int32)
                      % (_NACC - N_NODES))
    tail = jnp.broadcast_to(tail, (2, npad))
    return jnp.concatenate([ei.astype(jnp.int32), tail],
                           axis=1).reshape(2, _EROWS, _C)


def kernel(x_primal, x_dual, edge_index_pp, edge_index_dd, edge_index_pd,
           enc_p_W, enc_p_b, enc_d_W, enc_d_b, Wl, bl, Wr,
           dec_p_W, dec_p_b, dec_d_W, dec_d_b):
    epp = _pad_edges(edge_index_pp)
    edd = _pad_edges(edge_index_dd)
    epd = _pad_edges(edge_index_pd)

    zeros64 = jnp.zeros((_NACC, H), jnp.float32)
    zeros8 = jnp.zeros((_NACC, _CW), jnp.float32)
    ones8 = jnp.ones((_C, _CW), jnp.float32)

    x_st = jnp.stack([x_primal, x_dual]).reshape(2, N_NODES // 2, 2 * IN_DIM)
    encW = jnp.stack([enc_p_W, enc_d_W])
    encb = jnp.stack([enc_p_b, enc_d_b]).reshape(2, 1, H)
    h = _enc(x_st, encW, encb)

    decW = jnp.stack([dec_p_W, dec_d_W])
    decb = jnp.stack([dec_p_b, dec_d_b]).reshape(2, 1, OUT_DIM)

    def unpair(a):
        return a.reshape(2, _NACC, H)

    def pair(a):
        return a.reshape(2, _NPR, 2 * H)

    sA, sB, cA, cB = _sc_segsum0(unpair(h), epp, edd, epd,
                                 zeros64, zeros8, ones8)
    h = _combine(0, pair(sA), pair(sB), h, Wl, bl, Wr)
    sA, sB = _sc_segsum1(unpair(h), epp, edd, epd, cA, cB, zeros64)
    out = _combine(1, pair(sA), pair(sB), h, Wl, bl, Wr,
                   decW=decW, decb=decb)
    out = out.reshape(2, N_NODES, OUT_DIM)
    return (out[0], out[1])


# speedup vs baseline: 1.1278x; 1.0287x over previous
import functools

import jax
import jax.numpy as jnp
from jax import lax
from jax.experimental import pallas as pl
from jax.experimental.pallas import tpu as pltpu
from jax.experimental.pallas import tpu_sc as plsc

N_NODES = 10000
H = 64
E = 320000
OUT_DIM = 128
IN_DIM = 128

_NC = 2
_NS = 16
_C = 128
_EROWS = 2560
_RPT = _EROWS // _NS
_NACC = 10240
_ZROWS = _NACC // _NS
_CW = 8
_NB = 8


def _mesh():
    return plsc.VectorSubcoreMesh(core_axis_name="c", subcore_axis_name="s",
                                  num_cores=_NC, num_subcores=_NS)


def _segsum_body(with_counts, ich, h, epp, edd, epd, *refs):
    if with_counts:
        (zeros64, zeros8, ones8, oA, oB, cA, cB, sidx, didx, rows, onesv,
         cntv, nbuf, recips, acc, acc8, gsem, ssem, csem, isem) = refs
    else:
        (cAi, cBi, zeros64, oA, oB, sidx, didx, rows,
         cntv, nbuf, recips, acc, gsem, ssem, isem) = refs
    nch = _RPT // ich
    g_iters = ich // _NB
    c = lax.axis_index("c")
    s = lax.axis_index("s")
    rels = (
        (0, epp, 0, 1, 0, "A", 0),
        (0, epd, 1, 0, 1, "B", 0),
        (1, edd, 0, 1, 1, "A", 1),
        (1, epd, 0, 1, 0, "B", 1),
    )
    zoff = pl.multiple_of(s * _ZROWS, 8)
    eoff = pl.multiple_of(s * _RPT, 8)

    if with_counts:
        pltpu.sync_copy(ones8, onesv)

    for rc, earr, srow, drow, tslot, outn, oslot in rels:
        out = oA if outn == "A" else oB

        @pl.when(c == rc)
        def _zero():
            pltpu.sync_copy(zeros64.at[pl.ds(zoff, _ZROWS)],
                            acc.at[pl.ds(zoff, _ZROWS)])
            if with_counts:
                pltpu.sync_copy(zeros8.at[pl.ds(zoff, _ZROWS)],
                                acc8.at[pl.ds(zoff, _ZROWS)])

        plsc.subcore_barrier()

        @pl.when(c == rc)
        def _edges(earr=earr, srow=srow, drow=drow, tslot=tslot):
            table = h.at[tslot]

            def idx_load(ci, par, sem=None):
                coff = pl.multiple_of(eoff + ci * ich, 8)
                if sem is None:
                    pltpu.sync_copy(earr.at[srow, pl.ds(coff, ich)],
                                    sidx.at[par])
                    pltpu.sync_copy(earr.at[drow, pl.ds(coff, ich)],
                                    didx.at[par])
                else:
                    pltpu.async_copy(earr.at[srow, pl.ds(coff, ich)],
                                     sidx.at[par], sem)
                    pltpu.async_copy(earr.at[drow, pl.ds(coff, ich)],
                                     didx.at[par], sem)

            def idx_drain(ci, par):
                coff = pl.multiple_of(eoff + ci * ich, 8)
                pltpu.make_async_copy(earr.at[srow, pl.ds(coff, ich)],
                                      sidx.at[par], isem).wait()
                pltpu.make_async_copy(earr.at[drow, pl.ds(coff, ich)],
                                      didx.at[par], isem).wait()

            idx_load(0, 0)
            for b in range(_NB):
                pltpu.async_copy(table.at[sidx.at[0, b]], rows.at[b],
                                 gsem.at[b])
            idx_load(1, 1, isem)

            def do_chunk(ci, cur, nxt):
                has_next = ci + 1 < nch
                for g in range(g_iters):
                    for b in range(_NB):
                        k = g * _NB + b
                        pltpu.make_async_copy(table.at[sidx.at[cur, k]],
                                              rows.at[b], gsem.at[b]).wait()
                        pltpu.async_copy(rows.at[b], acc.at[didx.at[cur, k]],
                                         ssem.at[b], add=True)
                        if with_counts:
                            pltpu.async_copy(onesv, acc8.at[didx.at[cur, k]],
                                             csem.at[b], add=True)
                    if g == g_iters - 1:
                        @pl.when(has_next)
                        def _wait_idx():
                            idx_drain(ci + 1, nxt)
                    for b in range(_NB):
                        k = g * _NB + b
                        pltpu.make_async_copy(rows.at[b],
                                              acc.at[didx.at[cur, k]],
                                              ssem.at[b]).wait()
                        if with_counts:
                            pltpu.make_async_copy(
                                onesv, acc8.at[didx.at[cur, k]],
                                csem.at[b]).wait()
                        if g + 1 < g_iters:
                            pltpu.async_copy(
                                table.at[sidx.at[cur, (g + 1) * _NB + b]],
                                rows.at[b], gsem.at[b])
                        else:
                            @pl.when(has_next)
                            def _ring_next(b=b):
                                pltpu.async_copy(table.at[sidx.at[nxt, b]],
                                                 rows.at[b], gsem.at[b])

            def chunk_pair(cp, carry):
                for par in (0, 1):
                    ci = cp * 2 + par
                    do_chunk(ci, par, 1 - par)

                    @pl.when(ci + 2 < nch)
                    def _prefetch(ci=ci, par=par):
                        idx_load(ci + 2, par, isem)
                return carry

            lax.fori_loop(0, nch // 2, chunk_pair, 0)

        plsc.subcore_barrier()

        @pl.when(c == rc)
        def _normalize_and_copy_out(out=out, oslot=oslot, outn=outn):
            if with_counts:
                pltpu.sync_copy(acc8.at[pl.ds(zoff, _ZROWS)], cntv)
                cout = cA if outn == "A" else cB
                pltpu.sync_copy(acc8.at[pl.ds(zoff, _ZROWS)],
                                cout.at[oslot, pl.ds(zoff, _ZROWS)])
            else:
                cin = cAi if outn == "A" else cBi
                pltpu.sync_copy(cin.at[oslot, pl.ds(zoff, _ZROWS)], cntv)

            def recip16(g, carry):
                rowids = lax.iota(jnp.int32, 16) + g * 16
                cnt = plsc.load_gather(
                    cntv, [rowids, jnp.zeros((16,), jnp.int32)])
                recips[pl.ds(pl.multiple_of(g * 16, 8), 16)] = \
                    1.0 / jnp.maximum(cnt, 1.0)
                return carry

            lax.fori_loop(0, _ZROWS // 16, recip16, 0)

            def chunk_norm(ch, carry):
                roff = pl.multiple_of(ch * 64, 8)
                pltpu.sync_copy(acc.at[pl.ds(zoff + roff, 64)], nbuf)

                def row_norm(jj, carry2):
                    for u in range(4):
                        j = jj * 4 + u
                        r = plsc.load_gather(
                            recips, [jnp.full((16,), roff + j, jnp.int32)])
                        for c4 in range(4):
                            sl = pl.ds(c4 * 16, 16)
                            nbuf[j, sl] = nbuf[j, sl] * r
                    return carry2

                lax.fori_loop(0, 16, row_norm, 0)
                pltpu.sync_copy(nbuf,
                                out.at[oslot, pl.ds(zoff + roff, 64)])
                return carry

            lax.fori_loop(0, _ZROWS // 64, chunk_norm, 0)

        plsc.subcore_barrier()


@jax.jit
def _sc_segsum0(h, epp, edd, epd, zeros64, zeros8, ones8):
    ich = 16
    f = pl.kernel(
        functools.partial(_segsum_body, True, ich),
        out_type=[
            jax.ShapeDtypeStruct((2, _NACC, H), jnp.float32),
            jax.ShapeDtypeStruct((2, _NACC, H), jnp.float32),
            jax.ShapeDtypeStruct((2, _NACC, _CW), jnp.float32),
            jax.ShapeDtypeStruct((2, _NACC, _CW), jnp.float32),
        ],
        mesh=_mesh(),
        scratch_types=[
            pltpu.VMEM((2, ich, _C), jnp.int32),
            pltpu.VMEM((2, ich, _C), jnp.int32),
            pltpu.VMEM((_NB, _C, H), jnp.float32),
            pltpu.VMEM((_C, _CW), jnp.float32),
            pltpu.VMEM((_ZROWS, _CW), jnp.float32),
            pltpu.VMEM((64, H), jnp.float32),
            pltpu.VMEM((_ZROWS,), jnp.float32),
            pltpu.VMEM_SHARED((_NACC, H), jnp.float32),
            pltpu.VMEM_SHARED((_NACC, _CW), jnp.float32),
            pltpu.SemaphoreType.DMA((_NB,)),
            pltpu.SemaphoreType.DMA((_NB,)),
            pltpu.SemaphoreType.DMA((_NB,)),
            pltpu.SemaphoreType.DMA,
        ],
        compiler_params=pltpu.CompilerParams(use_tc_tiling_on_sc=False, needs_layout_passes=False),
    )
    return f(h, epp, edd, epd, zeros64, zeros8, ones8)


@jax.jit
def _sc_segsum1(h, epp, edd, epd, cA, cB, zeros64):
    ich = 16
    f = pl.kernel(
        functools.partial(_segsum_body, False, ich),
        out_type=[
            jax.ShapeDtypeStruct((2, _NACC, H), jnp.float32),
            jax.ShapeDtypeStruct((2, _NACC, H), jnp.float32),
        ],
        mesh=_mesh(),
        scratch_types=[
            pltpu.VMEM((2, ich, _C), jnp.int32),
            pltpu.VMEM((2, ich, _C), jnp.int32),
            pltpu.VMEM((_NB, _C, H), jnp.float32),
            pltpu.VMEM((_ZROWS, _CW), jnp.float32),
            pltpu.VMEM((64, H), jnp.float32),
            pltpu.VMEM((_ZROWS,), jnp.float32),
            pltpu.VMEM_SHARED((_NACC, H), jnp.float32),
            pltpu.SemaphoreType.DMA((_NB,)),
            pltpu.SemaphoreType.DMA((_NB,)),
            pltpu.SemaphoreType.DMA,
        ],
        compiler_params=pltpu.CompilerParams(use_tc_tiling_on_sc=False, needs_layout_passes=False),
    )
    return f(h, epp, edd, epd, cA, cB, zeros64)


_BN = 2000
_BP = _BN // 2
_NPR = _NACC // 2


def _enc(xp, W, b):
    def body(x_ref, w_ref, b_ref, o_ref):
        halves = []
        w16 = w_ref[0].astype(jnp.bfloat16)
        for lo in (0, IN_DIM):
            t = jnp.dot(x_ref[0][:, lo:lo + IN_DIM].astype(jnp.bfloat16),
                        w16, preferred_element_type=jnp.float32) + b_ref[0]
            halves.append(jnp.maximum(t, 0.0))
        o_ref[0] = jnp.concatenate(halves, axis=1)

    return pl.pallas_call(
        body,
        grid=(2, N_NODES // _BN),
        in_specs=[
            pl.BlockSpec((1, _BP, 2 * IN_DIM), lambda t, i: (t, i, 0)),
            pl.BlockSpec((1, IN_DIM, H), lambda t, i: (t, 0, 0)),
            pl.BlockSpec((1, 1, H), lambda t, i: (t, 0, 0)),
        ],
        out_specs=pl.BlockSpec((1, _BP, 2 * H), lambda t, i: (t, i, 0)),
        out_shape=jax.ShapeDtypeStruct((2, _NPR, 2 * H), jnp.float32),
    )(xp, W, b)


def _combine(layer, sA, sB, h, Wl, bl, Wr, decW=None, decb=None):
    decode = decW is not None
    l = layer

    def body(*refs):
        if decode:
            (sa, sb, hh, wa, wb, wra, wrb, bb, dw, db, o) = refs
        else:
            (sa, sb, hh, wa, wb, wra, wrb, bb, o) = refs
        tslot = pl.program_id(0)
        a = sa[0]
        bmsg = sb[0]
        wr = wra[0, 0] + wrb[0, 0]
        bias = bb[0, tslot] + bb[0, 3 - tslot]
        halves = []
        wa16 = wa[0, 0].astype(jnp.bfloat16)
        wb16 = wb[0, 0].astype(jnp.bfloat16)
        wr16 = wr.astype(jnp.bfloat16)
        dw16 = dw[0].astype(jnp.bfloat16) if decode else None
        for lo in (0, H):
            t = jnp.dot(a[:, lo:lo + H].astype(jnp.bfloat16), wa16,
                        preferred_element_type=jnp.float32)
            t = t + jnp.dot(bmsg[:, lo:lo + H].astype(jnp.bfloat16), wb16,
                            preferred_element_type=jnp.float32)
            t = t + jnp.dot(hh[0][:, lo:lo + H].astype(jnp.bfloat16), wr16,
                            preferred_element_type=jnp.float32)
            t = jnp.maximum(t + bias, 0.0)
            if decode:
                t = jnp.dot(t.astype(jnp.bfloat16), dw16,
                            preferred_element_type=jnp.float32) + db[0]
            halves.append(t)
        packed = jnp.concatenate(halves, axis=1)
        if decode:
            o[0] = packed.reshape(_BN, OUT_DIM)
        else:
            o[0] = packed

    in_specs = [
        pl.BlockSpec((1, _BP, 2 * H), lambda t, i: (t, i, 0)),
        pl.BlockSpec((1, _BP, 2 * H), lambda t, i: (t, i, 0)),
        pl.BlockSpec((1, _BP, 2 * H), lambda t, i: (t, i, 0)),
        pl.BlockSpec((1, 1, H, H), lambda t, i: (l, t, 0, 0)),
        pl.BlockSpec((1, 1, H, H), lambda t, i: (l, 3 - t, 0, 0)),
        pl.BlockSpec((1, 1, H, H), lambda t, i: (l, t, 0, 0)),
        pl.BlockSpec((1, 1, H, H), lambda t, i: (l, 3 - t, 0, 0)),
        pl.BlockSpec((1, 4, H), lambda t, i: (l, 0, 0)),
    ]
    args = [sA, sB, h, Wl, Wl, Wr, Wr, bl]
    if decode:
        in_specs += [
            pl.BlockSpec((1, H, OUT_DIM), lambda t, i: (t, 0, 0)),
            pl.BlockSpec((1, 1, OUT_DIM), lambda t, i: (t, 0, 0)),
        ]
        args += [decW, decb]
        out_spec = pl.BlockSpec((1, _BN, OUT_DIM), lambda t, i: (t, i, 0))
        out_shape = jax.ShapeDtypeStruct((2, N_NODES, OUT_DIM), jnp.float32)
    else:
        out_spec = pl.BlockSpec((1, _BP, 2 * H), lambda t, i: (t, i, 0))
        out_shape = jax.ShapeDtypeStruct((2, _NPR, 2 * H), jnp.float32)

    return pl.pallas_call(
        body,
        grid=(2, N_NODES // _BN),
        in_specs=in_specs,
        out_specs=out_spec,
        out_shape=out_shape,
    )(*args)


def _pad_edges(ei):
    npad = _EROWS * _C - E
    tail = N_NODES + (jnp.arange(npad, dtype=jnp.int32)
                      % (_NACC - N_NODES))
    tail = jnp.broadcast_to(tail, (2, npad))
    return jnp.concatenate([ei.astype(jnp.int32), tail],
                           axis=1).reshape(2, _EROWS, _C)


def kernel(x_primal, x_dual, edge_index_pp, edge_index_dd, edge_index_pd,
           enc_p_W, enc_p_b, enc_d_W, enc_d_b, Wl, bl, Wr,
           dec_p_W, dec_p_b, dec_d_W, dec_d_b):
    epp = _pad_edges(edge_index_pp)
    edd = _pad_edges(edge_index_dd)
    epd = _pad_edges(edge_index_pd)

    zeros64 = jnp.zeros((_NACC, H), jnp.float32)
    zeros8 = jnp.zeros((_NACC, _CW), jnp.float32)
    ones8 = jnp.ones((_C, _CW), jnp.float32)

    x_st = jnp.stack([x_primal, x_dual]).reshape(2, N_NODES // 2, 2 * IN_DIM)
    encW = jnp.stack([enc_p_W, enc_d_W])
    encb = jnp.stack([enc_p_b, enc_d_b]).reshape(2, 1, H)
    h = _enc(x_st, encW, encb)

    decW = jnp.stack([dec_p_W, dec_d_W])
    decb = jnp.stack([dec_p_b, dec_d_b]).reshape(2, 1, OUT_DIM)

    def unpair(a):
        return a.reshape(2, _NACC, H)

    def pair(a):
        return a.reshape(2, _NPR, 2 * H)

    sA, sB, cA, cB = _sc_segsum0(unpair(h), epp, edd, epd,
                                 zeros64, zeros8, ones8)
    h = _combine(0, pair(sA), pair(sB), h, Wl, bl, Wr)
    sA, sB = _sc_segsum1(unpair(h), epp, edd, epd, cA, cB, zeros64)
    out = _combine(1, pair(sA), pair(sB), h, Wl, bl, Wr,
                   decW=decW, decb=decb)
    return (out[0], out[1])


# speedup vs baseline: 1.1344x; 1.0058x over previous
import functools

import jax
import jax.numpy as jnp
from jax import lax
from jax.experimental import pallas as pl
from jax.experimental.pallas import tpu as pltpu
from jax.experimental.pallas import tpu_sc as plsc

N_NODES = 10000
H = 64
E = 320000
OUT_DIM = 128
IN_DIM = 128

_NC = 2
_NS = 16
_C = 128
_EROWS = 2560
_RPT = _EROWS // _NS
_NACC = 10240
_ZROWS = _NACC // _NS
_CW = 8
_NB = 8


def _mesh():
    return plsc.VectorSubcoreMesh(core_axis_name="c", subcore_axis_name="s",
                                  num_cores=_NC, num_subcores=_NS)


def _segsum_body(with_counts, ich, h, epp, edd, epd, *refs):
    if with_counts:
        (zeros64, zeros8, ones8, oA, oB, cA, cB, sidx, didx, rows, onesv,
         cntv, nbuf, recips, acc, acc8, gsem, ssem, csem, isem, nisem, nosem) = refs
    else:
        (cAi, cBi, zeros64, oA, oB, sidx, didx, rows,
         cntv, nbuf, recips, acc, gsem, ssem, isem, nisem, nosem) = refs
    nch = _RPT // ich
    g_iters = ich // _NB
    c = lax.axis_index("c")
    s = lax.axis_index("s")
    rels = (
        (0, epp, 0, 1, 0, "A", 0),
        (0, epd, 1, 0, 1, "B", 0),
        (1, edd, 0, 1, 1, "A", 1),
        (1, epd, 0, 1, 0, "B", 1),
    )
    zoff = pl.multiple_of(s * _ZROWS, 8)
    eoff = pl.multiple_of(s * _RPT, 8)

    if with_counts:
        pltpu.sync_copy(ones8, onesv)

    for rc, earr, srow, drow, tslot, outn, oslot in rels:
        out = oA if outn == "A" else oB

        @pl.when(c == rc)
        def _zero():
            pltpu.sync_copy(zeros64.at[pl.ds(zoff, _ZROWS)],
                            acc.at[pl.ds(zoff, _ZROWS)])
            if with_counts:
                pltpu.sync_copy(zeros8.at[pl.ds(zoff, _ZROWS)],
                                acc8.at[pl.ds(zoff, _ZROWS)])

        plsc.subcore_barrier()

        @pl.when(c == rc)
        def _edges(earr=earr, srow=srow, drow=drow, tslot=tslot):
            table = h.at[tslot]

            def idx_load(ci, par, sem=None):
                coff = pl.multiple_of(eoff + ci * ich, 8)
                if sem is None:
                    pltpu.sync_copy(earr.at[srow, pl.ds(coff, ich)],
                                    sidx.at[par])
                    pltpu.sync_copy(earr.at[drow, pl.ds(coff, ich)],
                                    didx.at[par])
                else:
                    pltpu.async_copy(earr.at[srow, pl.ds(coff, ich)],
                                     sidx.at[par], sem)
                    pltpu.async_copy(earr.at[drow, pl.ds(coff, ich)],
                                     didx.at[par], sem)

            def idx_drain(ci, par):
                coff = pl.multiple_of(eoff + ci * ich, 8)
                pltpu.make_async_copy(earr.at[srow, pl.ds(coff, ich)],
                                      sidx.at[par], isem).wait()
                pltpu.make_async_copy(earr.at[drow, pl.ds(coff, ich)],
                                      didx.at[par], isem).wait()

            idx_load(0, 0)
            for b in range(_NB):
                pltpu.async_copy(table.at[sidx.at[0, b]], rows.at[b],
                                 gsem.at[b])
            idx_load(1, 1, isem)

            def do_chunk(ci, cur, nxt):
                has_next = ci + 1 < nch
                for g in range(g_iters):
                    for b in range(_NB):
                        k = g * _NB + b
                        pltpu.make_async_copy(table.at[sidx.at[cur, k]],
                                              rows.at[b], gsem.at[b]).wait()
                        pltpu.async_copy(rows.at[b], acc.at[didx.at[cur, k]],
                                         ssem.at[b], add=True)
                        if with_counts:
                            pltpu.async_copy(onesv, acc8.at[didx.at[cur, k]],
                                             csem.at[b], add=True)
                    if g == g_iters - 1:
                        @pl.when(has_next)
                        def _wait_idx():
                            idx_drain(ci + 1, nxt)
                    for b in range(_NB):
                        k = g * _NB + b
                        pltpu.make_async_copy(rows.at[b],
                                              acc.at[didx.at[cur, k]],
                                              ssem.at[b]).wait()
                        if with_counts:
                            pltpu.make_async_copy(
                                onesv, acc8.at[didx.at[cur, k]],
                                csem.at[b]).wait()
                        if g + 1 < g_iters:
                            pltpu.async_copy(
                                table.at[sidx.at[cur, (g + 1) * _NB + b]],
                                rows.at[b], gsem.at[b])
                        else:
                            @pl.when(has_next)
                            def _ring_next(b=b):
                                pltpu.async_copy(table.at[sidx.at[nxt, b]],
                                                 rows.at[b], gsem.at[b])

            def chunk_pair(cp, carry):
                for par in (0, 1):
                    ci = cp * 2 + par
                    do_chunk(ci, par, 1 - par)

                    @pl.when(ci + 2 < nch)
                    def _prefetch(ci=ci, par=par):
                        idx_load(ci + 2, par, isem)
                return carry

            lax.fori_loop(0, nch // 2, chunk_pair, 0)

        plsc.subcore_barrier()

        @pl.when(c == rc)
        def _normalize_and_copy_out(out=out, oslot=oslot, outn=outn):
            if with_counts:
                pltpu.sync_copy(acc8.at[pl.ds(zoff, _ZROWS)], cntv)
                cout = cA if outn == "A" else cB
                pltpu.sync_copy(acc8.at[pl.ds(zoff, _ZROWS)],
                                cout.at[oslot, pl.ds(zoff, _ZROWS)])
            else:
                cin = cAi if outn == "A" else cBi
                pltpu.sync_copy(cin.at[oslot, pl.ds(zoff, _ZROWS)], cntv)

            def recip16(g, carry):
                rowids = lax.iota(jnp.int32, 16) + g * 16
                cnt = plsc.load_gather(
                    cntv, [rowids, jnp.zeros((16,), jnp.int32)])
                recips[pl.ds(pl.multiple_of(g * 16, 8), 16)] = \
                    1.0 / jnp.maximum(cnt, 1.0)
                return carry

            lax.fori_loop(0, _ZROWS // 16, recip16, 0)

            ncn = _ZROWS // 32

            def cin(ch, par):
                return pltpu.make_async_copy(
                    acc.at[pl.ds(zoff + pl.multiple_of(ch * 32, 8), 32)],
                    nbuf.at[par], nisem.at[par])

            def cout(ch, par):
                return pltpu.make_async_copy(
                    nbuf.at[par],
                    out.at[oslot, pl.ds(zoff + pl.multiple_of(ch * 32, 8),
                                        32)],
                    nosem.at[par])

            cin(0, 0).start()

            def norm_pair(cp, carry):
                for par in (0, 1):
                    ch = cp * 2 + par

                    @pl.when(ch >= 1)
                    def _free(ch=ch, par=par):
                        cout(ch - 1, 1 - par).wait()

                    @pl.when(ch + 1 < ncn)
                    def _next_in(ch=ch, par=par):
                        cin(ch + 1, 1 - par).start()

                    cin(ch, par).wait()

                    def row_norm(jj, carry2, ch=ch, par=par):
                        for u in range(4):
                            j = jj * 4 + u
                            r = plsc.load_gather(
                                recips,
                                [jnp.full((16,), ch * 32 + j, jnp.int32)])
                            for c4 in range(4):
                                sl = pl.ds(c4 * 16, 16)
                                nbuf[par, j, sl] = nbuf[par, j, sl] * r
                        return carry2

                    lax.fori_loop(0, 8, row_norm, 0)
                    cout(ch, par).start()
                return carry

            lax.fori_loop(0, ncn // 2, norm_pair, 0)
            cout(ncn - 1, 1).wait()

        plsc.subcore_barrier()


@jax.jit
def _sc_segsum0(h, epp, edd, epd, zeros64, zeros8, ones8):
    ich = 16
    f = pl.kernel(
        functools.partial(_segsum_body, True, ich),
        out_type=[
            jax.ShapeDtypeStruct((2, _NACC, H), jnp.float32),
            jax.ShapeDtypeStruct((2, _NACC, H), jnp.float32),
            jax.ShapeDtypeStruct((2, _NACC, _CW), jnp.float32),
            jax.ShapeDtypeStruct((2, _NACC, _CW), jnp.float32),
        ],
        mesh=_mesh(),
        scratch_types=[
            pltpu.VMEM((2, ich, _C), jnp.int32),
            pltpu.VMEM((2, ich, _C), jnp.int32),
            pltpu.VMEM((_NB, _C, H), jnp.float32),
            pltpu.VMEM((_C, _CW), jnp.float32),
            pltpu.VMEM((_ZROWS, _CW), jnp.float32),
            pltpu.VMEM((2, 32, H), jnp.float32),
            pltpu.VMEM((_ZROWS,), jnp.float32),
            pltpu.VMEM_SHARED((_NACC, H), jnp.float32),
            pltpu.VMEM_SHARED((_NACC, _CW), jnp.float32),
            pltpu.SemaphoreType.DMA((_NB,)),
            pltpu.SemaphoreType.DMA((_NB,)),
            pltpu.SemaphoreType.DMA((_NB,)),
            pltpu.SemaphoreType.DMA,
            pltpu.SemaphoreType.DMA((2,)),
            pltpu.SemaphoreType.DMA((2,)),
        ],
        compiler_params=pltpu.CompilerParams(use_tc_tiling_on_sc=False, needs_layout_passes=False),
    )
    return f(h, epp, edd, epd, zeros64, zeros8, ones8)


@jax.jit
def _sc_segsum1(h, epp, edd, epd, cA, cB, zeros64):
    ich = 16
    f = pl.kernel(
        functools.partial(_segsum_body, False, ich),
        out_type=[
            jax.ShapeDtypeStruct((2, _NACC, H), jnp.float32),
            jax.ShapeDtypeStruct((2, _NACC, H), jnp.float32),
        ],
        mesh=_mesh(),
        scratch_types=[
            pltpu.VMEM((2, ich, _C), jnp.int32),
            pltpu.VMEM((2, ich, _C), jnp.int32),
            pltpu.VMEM((_NB, _C, H), jnp.float32),
            pltpu.VMEM((_ZROWS, _CW), jnp.float32),
            pltpu.VMEM((2, 32, H), jnp.float32),
            pltpu.VMEM((_ZROWS,), jnp.float32),
            pltpu.VMEM_SHARED((_NACC, H), jnp.float32),
            pltpu.SemaphoreType.DMA((_NB,)),
            pltpu.SemaphoreType.DMA((_NB,)),
            pltpu.SemaphoreType.DMA,
            pltpu.SemaphoreType.DMA((2,)),
            pltpu.SemaphoreType.DMA((2,)),
        ],
        compiler_params=pltpu.CompilerParams(use_tc_tiling_on_sc=False, needs_layout_passes=False),
    )
    return f(h, epp, edd, epd, cA, cB, zeros64)


_BN = 2000
_BP = _BN // 2
_NPR = _NACC // 2


def _enc(xp, W, b):
    def body(x_ref, w_ref, b_ref, o_ref):
        halves = []
        w16 = w_ref[0].astype(jnp.bfloat16)
        for lo in (0, IN_DIM):
            t = jnp.dot(x_ref[0][:, lo:lo + IN_DIM].astype(jnp.bfloat16),
                        w16, preferred_element_type=jnp.float32) + b_ref[0]
            halves.append(jnp.maximum(t, 0.0))
        o_ref[0] = jnp.concatenate(halves, axis=1)

    return pl.pallas_call(
        body,
        grid=(2, N_NODES // _BN),
        in_specs=[
            pl.BlockSpec((1, _BP, 2 * IN_DIM), lambda t, i: (t, i, 0)),
            pl.BlockSpec((1, IN_DIM, H), lambda t, i: (t, 0, 0)),
            pl.BlockSpec((1, 1, H), lambda t, i: (t, 0, 0)),
        ],
        out_specs=pl.BlockSpec((1, _BP, 2 * H), lambda t, i: (t, i, 0)),
        out_shape=jax.ShapeDtypeStruct((2, _NPR, 2 * H), jnp.float32),
    )(xp, W, b)


def _combine(layer, sA, sB, h, Wl, bl, Wr, decW=None, decb=None):
    decode = decW is not None
    l = layer

    def body(*refs):
        if decode:
            (sa, sb, hh, wa, wb, wra, wrb, bb, dw, db, o) = refs
        else:
            (sa, sb, hh, wa, wb, wra, wrb, bb, o) = refs
        tslot = pl.program_id(0)
        a = sa[0]
        bmsg = sb[0]
        wr = wra[0, 0] + wrb[0, 0]
        bias = bb[0, tslot] + bb[0, 3 - tslot]
        halves = []
        wa16 = wa[0, 0].astype(jnp.bfloat16)
        wb16 = wb[0, 0].astype(jnp.bfloat16)
        wr16 = wr.astype(jnp.bfloat16)
        dw16 = dw[0].astype(jnp.bfloat16) if decode else None
        for lo in (0, H):
            t = jnp.dot(a[:, lo:lo + H].astype(jnp.bfloat16), wa16,
                        preferred_element_type=jnp.float32)
            t = t + jnp.dot(bmsg[:, lo:lo + H].astype(jnp.bfloat16), wb16,
                            preferred_element_type=jnp.float32)
            t = t + jnp.dot(hh[0][:, lo:lo + H].astype(jnp.bfloat16), wr16,
                            preferred_element_type=jnp.float32)
            t = jnp.maximum(t + bias, 0.0)
            if decode:
                t = jnp.dot(t.astype(jnp.bfloat16), dw16,
                            preferred_element_type=jnp.float32) + db[0]
            halves.append(t)
        packed = jnp.concatenate(halves, axis=1)
        if decode:
            o[0] = packed.reshape(_BN, OUT_DIM)
        else:
            o[0] = packed

    in_specs = [
        pl.BlockSpec((1, _BP, 2 * H), lambda t, i: (t, i, 0)),
        pl.BlockSpec((1, _BP, 2 * H), lambda t, i: (t, i, 0)),
        pl.BlockSpec((1, _BP, 2 * H), lambda t, i: (t, i, 0)),
        pl.BlockSpec((1, 1, H, H), lambda t, i: (l, t, 0, 0)),
        pl.BlockSpec((1, 1, H, H), lambda t, i: (l, 3 - t, 0, 0)),
        pl.BlockSpec((1, 1, H, H), lambda t, i: (l, t, 0, 0)),
        pl.BlockSpec((1, 1, H, H), lambda t, i: (l, 3 - t, 0, 0)),
        pl.BlockSpec((1, 4, H), lambda t, i: (l, 0, 0)),
    ]
    args = [sA, sB, h, Wl, Wl, Wr, Wr, bl]
    if decode:
        in_specs += [
            pl.BlockSpec((1, H, OUT_DIM), lambda t, i: (t, 0, 0)),
            pl.BlockSpec((1, 1, OUT_DIM), lambda t, i: (t, 0, 0)),
        ]
        args += [decW, decb]
        out_spec = pl.BlockSpec((1, _BN, OUT_DIM), lambda t, i: (t, i, 0))
        out_shape = jax.ShapeDtypeStruct((2, N_NODES, OUT_DIM), jnp.float32)
    else:
        out_spec = pl.BlockSpec((1, _BP, 2 * H), lambda t, i: (t, i, 0))
        out_shape = jax.ShapeDtypeStruct((2, _NPR, 2 * H), jnp.float32)

    return pl.pallas_call(
        body,
        grid=(2, N_NODES // _BN),
        in_specs=in_specs,
        out_specs=out_spec,
        out_shape=out_shape,
    )(*args)


def _pad_edges(ei):
    npad = _EROWS * _C - E
    tail = N_NODES + (jnp.arange(npad, dtype=jnp.int32)
                      % (_NACC - N_NODES))
    tail = jnp.broadcast_to(tail, (2, npad))
    return jnp.concatenate([ei.astype(jnp.int32), tail],
                           axis=1).reshape(2, _EROWS, _C)


def kernel(x_primal, x_dual, edge_index_pp, edge_index_dd, edge_index_pd,
           enc_p_W, enc_p_b, enc_d_W, enc_d_b, Wl, bl, Wr,
           dec_p_W, dec_p_b, dec_d_W, dec_d_b):
    epp = _pad_edges(edge_index_pp)
    edd = _pad_edges(edge_index_dd)
    epd = _pad_edges(edge_index_pd)

    zeros64 = jnp.zeros((_NACC, H), jnp.float32)
    zeros8 = jnp.zeros((_NACC, _CW), jnp.float32)
    ones8 = jnp.ones((_C, _CW), jnp.float32)

    x_st = jnp.stack([x_primal, x_dual]).reshape(2, N_NODES // 2, 2 * IN_DIM)
    encW = jnp.stack([enc_p_W, enc_d_W])
    encb = jnp.stack([enc_p_b, enc_d_b]).reshape(2, 1, H)
    h = _enc(x_st, encW, encb)

    decW = jnp.stack([dec_p_W, dec_d_W])
    decb = jnp.stack([dec_p_b, dec_d_b]).reshape(2, 1, OUT_DIM)

    def unpair(a):
        return a.reshape(2, _NACC, H)

    def pair(a):
        return a.reshape(2, _NPR, 2 * H)

    sA, sB, cA, cB = _sc_segsum0(unpair(h), epp, edd, epd,
                                 zeros64, zeros8, ones8)
    h = _combine(0, pair(sA), pair(sB), h, Wl, bl, Wr)
    sA, sB = _sc_segsum1(unpair(h), epp, edd, epd, cA, cB, zeros64)
    out = _combine(1, pair(sA), pair(sB), h, Wl, bl, Wr,
                   decW=decW, decb=decb)
    return (out[0], out[1])
